# baseline (device time: 440442 ns/iter reference)
import jax
import jax.numpy as jnp
from jax import lax
from jax.experimental import pallas as pl
from jax.experimental.pallas import tpu as pltpu

M_FULL = 8192
D = 4096
M_OUT = 4096
H = 2048

SIZES = [128] * 14 + [64, 64, 32, 32, 16, 16, 16, 16]
OFFS = [sum(SIZES[:i]) for i in range(len(SIZES))]
NCH = len(SIZES)
CMAX = max(SIZES)
PREISSUE = 12


def kernel(partial, gamma):
    gamma2 = gamma.reshape(1, D)

    def body(partial_ref, gamma_ref, out_ref, brcv, abuf, bbuf, obuf,
             s1s, s1r, s2s, s2r, cp_a, cp_b, cp_o):
        my_x = lax.axis_index("x")
        my_y = lax.axis_index("y")
        base = my_y * M_OUT
        doff = my_x * H
        xoff = (1 - my_x) * H

        def start_send1(k):
            r = pltpu.make_async_remote_copy(
                src_ref=partial_ref.at[
                    0, pl.ds((1 - my_y) * M_OUT + doff + OFFS[k], SIZES[k]), :],
                dst_ref=brcv.at[pl.ds(doff + OFFS[k], SIZES[k]), :],
                send_sem=s1s.at[k],
                recv_sem=s1r.at[k],
                device_id=(my_x, 1 - my_y),
                device_id_type=pl.DeviceIdType.MESH,
            )
            r.start()
            return r

        rdma1 = [start_send1(k) for k in range(PREISSUE)]

        sched = []
        for k in range(NCH):
            sched.append(("D", k))
            if k >= 1:
                sched.append(("X", k - 1))
        sched.append(("X", NCH - 1))
        n = len(sched)

        def off_of(kind, k):
            return (doff if kind == "D" else xoff) + OFFS[k]

        rdma2 = []
        in_desc = [None, None]
        out_desc = [None, None]

        for i in range(n + 1):
            if i < n:
                kind, k = sched[i]
                if kind == "D":
                    if k + PREISSUE < NCH:
                        rdma1.append(start_send1(k + PREISSUE))
                    rdma1[k].wait_recv()
                    r = pltpu.make_async_remote_copy(
                        src_ref=brcv.at[pl.ds(doff + OFFS[k], SIZES[k]), :],
                        dst_ref=brcv.at[pl.ds(doff + OFFS[k], SIZES[k]), :],
                        send_sem=s2s.at[k],
                        recv_sem=s2r.at[k],
                        device_id=(1 - my_x, my_y),
                        device_id_type=pl.DeviceIdType.MESH,
                    )
                    r.start()
                    rdma2.append(r)
                else:
                    rdma2[k].wait_recv()
                s = i % 2
                off = off_of(kind, k)
                sz = SIZES[k]
                ca = pltpu.make_async_copy(
                    partial_ref.at[0, pl.ds(base + off, sz), :],
                    abuf.at[s, pl.ds(0, sz), :], cp_a.at[s])
                ca.start()
                cb = pltpu.make_async_copy(
                    brcv.at[pl.ds(off, sz), :],
                    bbuf.at[s, pl.ds(0, sz), :], cp_b.at[s])
                cb.start()
                in_desc[s] = (ca, cb)
            if i >= 1:
                kindj, kj = sched[i - 1]
                s = (i - 1) % 2
                szj = SIZES[kj]
                ca, cb = in_desc[s]
                ca.wait()
                cb.wait()
                yv = abuf[s, 0:szj, :] + bbuf[s, 0:szj, :]
                ms = jnp.mean(yv * yv, axis=1, keepdims=True)
                if out_desc[s] is not None:
                    out_desc[s].wait()
                obuf[s, 0:szj, :] = yv * lax.rsqrt(ms + 1e-6) * gamma_ref[...]
                co = pltpu.make_async_copy(
                    obuf.at[s, pl.ds(0, szj), :],
                    out_ref.at[pl.ds(off_of(kindj, kj), szj), :],
                    cp_o.at[s])
                co.start()
                out_desc[s] = co

        out_desc[(n - 1) % 2].wait()
        out_desc[n % 2].wait()
        for k in range(NCH):
            rdma1[k].wait_send()
            rdma2[k].wait_send()

    return pl.pallas_call(
        body,
        out_shape=[
            jax.ShapeDtypeStruct((M_OUT, D), jnp.float32),
            jax.ShapeDtypeStruct((M_OUT, D), jnp.float32),
        ],
        in_specs=[
            pl.BlockSpec(memory_space=pl.MemorySpace.ANY),
            pl.BlockSpec(memory_space=pltpu.MemorySpace.VMEM),
        ],
        out_specs=[
            pl.BlockSpec(memory_space=pl.MemorySpace.ANY),
            pl.BlockSpec(memory_space=pl.MemorySpace.ANY),
        ],
        scratch_shapes=[
            pltpu.VMEM((2, CMAX, D), jnp.float32),
            pltpu.VMEM((2, CMAX, D), jnp.float32),
            pltpu.VMEM((2, CMAX, D), jnp.float32),
            pltpu.SemaphoreType.DMA((NCH,)),
            pltpu.SemaphoreType.DMA((NCH,)),
            pltpu.SemaphoreType.DMA((NCH,)),
            pltpu.SemaphoreType.DMA((NCH,)),
            pltpu.SemaphoreType.DMA((2,)),
            pltpu.SemaphoreType.DMA((2,)),
            pltpu.SemaphoreType.DMA((2,)),
        ],
    )(partial, gamma2)[0]
